# Initial kernel scaffold; baseline (speedup 1.0000x reference)
import functools

import jax
import jax.numpy as jnp
from jax.experimental import pallas as pl

N = 10000
E = 320000
EP = 330000


def _mm_kernel(x_ref, w_ref, o_ref):
    o_ref[...] = jnp.dot(x_ref[...], w_ref[...], preferred_element_type=jnp.float32)


def _matmul(x, w):
    return pl.pallas_call(
        _mm_kernel,
        out_shape=jax.ShapeDtypeStruct((x.shape[0], w.shape[1]), jnp.float32),
    )(x, w)


def _edge_e_kernel(s_ref, d_ref, o_ref):
    e = s_ref[...] + d_ref[...]
    o_ref[...] = jnp.where(e > 0, e, 0.2 * e)


def _edge_exp_kernel(e_ref, m_ref, o_ref):
    o_ref[...] = jnp.exp(e_ref[...] - m_ref[...])


def _edge_alpha_kernel(z_ref, dn_ref, o_ref):
    o_ref[...] = z_ref[...] / dn_ref[...]


def _ew1(kernel_fn, a, b):
    n = a.shape[0]
    p = ((n + 1023) // 1024) * 1024
    ap = jnp.pad(a, (0, p - n))
    bp = jnp.pad(b, (0, p - n))
    out = pl.pallas_call(
        kernel_fn, out_shape=jax.ShapeDtypeStruct((p,), jnp.float32)
    )(ap, bp)
    return out[:n]


def _upd_kernel(al_ref, h_ref, o_ref):
    o_ref[...] = al_ref[...][:, None] * h_ref[...]


def _upd(alpha, hsrc):
    n, f = hsrc.shape
    p = ((n + 255) // 256) * 256
    ap = jnp.pad(alpha, (0, p - n))
    hp = jnp.pad(hsrc, ((0, p - n), (0, 0)))
    out = pl.pallas_call(
        _upd_kernel, out_shape=jax.ShapeDtypeStruct((p, f), jnp.float32)
    )(ap, hp)
    return out[:n]


def _act_leaky_kernel(slope, o_ref, b_ref, y_ref):
    t = o_ref[...] + b_ref[...]
    y_ref[...] = jnp.where(t > 0, t, slope * t)


def _act_bias_kernel(o_ref, b_ref, y_ref):
    y_ref[...] = o_ref[...] + b_ref[...]


def _act(out, b, slope):
    n, f = out.shape
    if slope is None:
        body = _act_bias_kernel
    else:
        body = functools.partial(_act_leaky_kernel, slope)
    return pl.pallas_call(
        body, out_shape=jax.ShapeDtypeStruct((n, f), jnp.float32)
    )(out, jnp.broadcast_to(b[None, :], (n, f)))


def _gat_layer(x, src, dst, W, a_src, a_dst, b, act_slope):
    h = _matmul(x, W)
    al_s = (h * a_src).sum(axis=-1)
    al_d = (h * a_dst).sum(axis=-1)
    e = _ew1(_edge_e_kernel, al_s[src], al_d[dst])
    e_max = jax.ops.segment_max(e, dst, num_segments=N)
    e_exp = _ew1(_edge_exp_kernel, e, e_max[dst])
    denom = jax.ops.segment_sum(e_exp, dst, num_segments=N)
    alpha = _ew1(_edge_alpha_kernel, e_exp, denom[dst])
    upd = _upd(alpha, h[src])
    out = jax.ops.segment_sum(upd, dst, num_segments=N)
    return _act(out, b, act_slope)


def kernel(x, edge_index, W1, a_src1, a_dst1, b1, W2, a_src2, a_dst2, b2):
    loop = jnp.arange(N, dtype=edge_index.dtype)
    src = jnp.concatenate([edge_index[0], loop])
    dst = jnp.concatenate([edge_index[1], loop])

    x1 = _gat_layer(x, src, dst, W1, a_src1, a_dst1, b1, 0.01)
    x2 = _gat_layer(x1, src, dst, W2, a_src2, a_dst2, b2, None)

    value = (x2[edge_index[0]] * x2[edge_index[1]]).sum(axis=1)
    k_homo = int(E * 0.6)
    k_het = E - k_homo
    _, topk_homo = jax.lax.top_k(value, k_homo)
    _, topk_het = jax.lax.top_k(-value, k_het)
    return edge_index[:, topk_homo], edge_index[:, topk_het], x2


# speedup vs baseline: 1.1621x; 1.1621x over previous
import functools

import jax
import jax.numpy as jnp
from jax.experimental import pallas as pl

N = 10000
E = 320000
EP = 330000


def _mm_kernel(x_ref, w_ref, o_ref):
    o_ref[...] = jnp.dot(x_ref[...], w_ref[...], preferred_element_type=jnp.float32)


def _matmul(x, w):
    return pl.pallas_call(
        _mm_kernel,
        out_shape=jax.ShapeDtypeStruct((x.shape[0], w.shape[1]), jnp.float32),
    )(x, w)


def _edge_e_kernel(s_ref, d_ref, o_ref):
    e = s_ref[...] + d_ref[...]
    o_ref[...] = jnp.where(e > 0, e, 0.2 * e)


def _edge_exp_kernel(e_ref, m_ref, o_ref):
    o_ref[...] = jnp.exp(e_ref[...] - m_ref[...])


def _edge_alpha_kernel(z_ref, dn_ref, o_ref):
    o_ref[...] = z_ref[...] / dn_ref[...]


def _ew1(kernel_fn, a, b):
    n = a.shape[0]
    p = ((n + 1023) // 1024) * 1024
    ap = jnp.pad(a, (0, p - n))
    bp = jnp.pad(b, (0, p - n))
    out = pl.pallas_call(
        kernel_fn, out_shape=jax.ShapeDtypeStruct((p,), jnp.float32)
    )(ap, bp)
    return out[:n]


def _upd_kernel(al_ref, h_ref, o_ref):
    o_ref[...] = al_ref[...][:, None] * h_ref[...]


def _upd(alpha, hsrc):
    n, f = hsrc.shape
    blk = 8192
    p = ((n + blk - 1) // blk) * blk
    ap = jnp.pad(alpha, (0, p - n))
    hp = jnp.pad(hsrc, ((0, p - n), (0, 0)))
    out = pl.pallas_call(
        _upd_kernel,
        grid=(p // blk,),
        in_specs=[
            pl.BlockSpec((blk,), lambda i: (i,)),
            pl.BlockSpec((blk, f), lambda i: (i, 0)),
        ],
        out_specs=pl.BlockSpec((blk, f), lambda i: (i, 0)),
        out_shape=jax.ShapeDtypeStruct((p, f), jnp.float32),
    )(ap, hp)
    return out[:n]


def _act_leaky_kernel(slope, o_ref, b_ref, y_ref):
    t = o_ref[...] + b_ref[...]
    y_ref[...] = jnp.where(t > 0, t, slope * t)


def _act_bias_kernel(o_ref, b_ref, y_ref):
    y_ref[...] = o_ref[...] + b_ref[...]


def _act(out, b, slope):
    n, f = out.shape
    if slope is None:
        body = _act_bias_kernel
    else:
        body = functools.partial(_act_leaky_kernel, slope)
    return pl.pallas_call(
        body, out_shape=jax.ShapeDtypeStruct((n, f), jnp.float32)
    )(out, jnp.broadcast_to(b[None, :], (n, f)))


def _gat_layer(x, src, dst, W, a_src, a_dst, b, act_slope):
    h = _matmul(x, W)
    al_s = (h * a_src).sum(axis=-1)
    al_d = (h * a_dst).sum(axis=-1)
    e = _ew1(_edge_e_kernel, al_s[src], al_d[dst])
    e_max = jax.ops.segment_max(e, dst, num_segments=N)
    e_exp = _ew1(_edge_exp_kernel, e, e_max[dst])
    denom = jax.ops.segment_sum(e_exp, dst, num_segments=N)
    alpha = _ew1(_edge_alpha_kernel, e_exp, denom[dst])
    upd = _upd(alpha, h[src])
    out = jax.ops.segment_sum(upd, dst, num_segments=N)
    return _act(out, b, act_slope)


def kernel(x, edge_index, W1, a_src1, a_dst1, b1, W2, a_src2, a_dst2, b2):
    loop = jnp.arange(N, dtype=edge_index.dtype)
    src = jnp.concatenate([edge_index[0], loop])
    dst = jnp.concatenate([edge_index[1], loop])

    x1 = _gat_layer(x, src, dst, W1, a_src1, a_dst1, b1, 0.01)
    x2 = _gat_layer(x1, src, dst, W2, a_src2, a_dst2, b2, None)

    value = (x2[edge_index[0]] * x2[edge_index[1]]).sum(axis=1)
    k_homo = int(E * 0.6)
    k_het = E - k_homo
    _, topk_homo = jax.lax.top_k(value, k_homo)
    _, topk_het = jax.lax.top_k(-value, k_het)
    return edge_index[:, topk_homo], edge_index[:, topk_het], x2


# speedup vs baseline: 2.7740x; 2.3870x over previous
import functools

import jax
import jax.numpy as jnp
from jax import lax
from jax.experimental import pallas as pl
from jax.experimental.pallas import tpu as pltpu
from jax.experimental.pallas import tpu_sc as plsc

N = 10000
E = 320000
EP = 330000
NW = 32
EPAD = 330240
BPW = EPAD // NW


def _sc_gather_call(tables, idxs):
    k = len(tables)
    mesh = plsc.VectorSubcoreMesh(core_axis_name="c", subcore_axis_name="s")
    dtypes = [t.dtype for t in tables]

    @functools.partial(
        pl.kernel,
        out_type=tuple(jax.ShapeDtypeStruct((EPAD,), dt) for dt in dtypes),
        mesh=mesh,
        scratch_types=[pltpu.VMEM((BPW,), jnp.int32),
                       pltpu.VMEM((BPW,), jnp.float32),
                       pltpu.SemaphoreType.DMA],
    )
    def _k(*refs):
        tab_hbm = refs[:k]
        idx_hbm = refs[k:2 * k]
        out_hbm = refs[2 * k:3 * k]
        idx_v, out_v, sem = refs[3 * k:]
        wid = lax.axis_index("s") * 2 + lax.axis_index("c")
        base = wid * BPW
        ch = 1032
        for p in range(k):
            pltpu.sync_copy(idx_hbm[p].at[pl.ds(base, BPW)], idx_v)
            cps = [pltpu.async_copy(tab_hbm[p].at[idx_v.at[pl.ds(j * ch, ch)]],
                                    out_v.at[pl.ds(j * ch, ch)], sem)
                   for j in range(BPW // ch)]
            for c in cps:
                c.wait()
            pltpu.sync_copy(out_v, out_hbm[p].at[pl.ds(base, BPW)])

    outs = _k(*tables, *idxs)
    return (outs,) if not isinstance(outs, (tuple, list)) else tuple(outs)


def _sc_gather(pairs):
    tables = [t for t, _ in pairs]
    idxs = [i for _, i in pairs]
    outs = _sc_gather_call(tables, idxs)
    return [o[:EP] for o in outs]


def _mm_kernel(x_ref, w_ref, o_ref):
    o_ref[...] = jnp.dot(x_ref[...], w_ref[...], preferred_element_type=jnp.float32)


def _matmul(x, w):
    return pl.pallas_call(
        _mm_kernel,
        out_shape=jax.ShapeDtypeStruct((x.shape[0], w.shape[1]), jnp.float32),
    )(x, w)


def _edge_e_kernel(s_ref, d_ref, o_ref):
    e = s_ref[...] + d_ref[...]
    o_ref[...] = jnp.where(e > 0, e, 0.2 * e)


def _edge_exp_kernel(e_ref, m_ref, o_ref):
    o_ref[...] = jnp.exp(e_ref[...] - m_ref[...])


def _edge_alpha_kernel(z_ref, dn_ref, o_ref):
    o_ref[...] = z_ref[...] / dn_ref[...]


def _ew1(kernel_fn, a, b):
    n = a.shape[0]
    p = ((n + 1023) // 1024) * 1024
    ap = jnp.pad(a, (0, p - n))
    bp = jnp.pad(b, (0, p - n))
    out = pl.pallas_call(
        kernel_fn, out_shape=jax.ShapeDtypeStruct((p,), jnp.float32)
    )(ap, bp)
    return out[:n]


def _upd_kernel(al_ref, h_ref, o_ref):
    o_ref[...] = al_ref[...][:, None] * h_ref[...]


def _upd(alpha, hsrc):
    n, f = hsrc.shape
    blk = 8192
    p = ((n + blk - 1) // blk) * blk
    ap = jnp.pad(alpha, (0, p - n))
    hp = jnp.pad(hsrc, ((0, p - n), (0, 0)))
    out = pl.pallas_call(
        _upd_kernel,
        grid=(p // blk,),
        in_specs=[
            pl.BlockSpec((blk,), lambda i: (i,)),
            pl.BlockSpec((blk, f), lambda i: (i, 0)),
        ],
        out_specs=pl.BlockSpec((blk, f), lambda i: (i, 0)),
        out_shape=jax.ShapeDtypeStruct((p, f), jnp.float32),
    )(ap, hp)
    return out[:n]


def _act_leaky_kernel(slope, o_ref, b_ref, y_ref):
    t = o_ref[...] + b_ref[...]
    y_ref[...] = jnp.where(t > 0, t, slope * t)


def _act_bias_kernel(o_ref, b_ref, y_ref):
    y_ref[...] = o_ref[...] + b_ref[...]


def _act(out, b, slope):
    n, f = out.shape
    if slope is None:
        body = _act_bias_kernel
    else:
        body = functools.partial(_act_leaky_kernel, slope)
    return pl.pallas_call(
        body, out_shape=jax.ShapeDtypeStruct((n, f), jnp.float32)
    )(out, jnp.broadcast_to(b[None, :], (n, f)))


def _gat_layer(x, idx, W, a_src, a_dst, b, act_slope):
    ssrc, sdst, ssrc_pad, sdst_pad, src_pad, dst_pad, dst = idx
    h = _matmul(x, W)
    al_s = (h * a_src).sum(axis=-1)
    al_d = (h * a_dst).sum(axis=-1)
    g_s_s, g_d_s, g_s_u, g_d_u = _sc_gather(
        [(al_s, ssrc_pad), (al_d, sdst_pad), (al_s, src_pad), (al_d, dst_pad)])
    e_s = _ew1(_edge_e_kernel, g_s_s, g_d_s)
    e_u = _ew1(_edge_e_kernel, g_s_u, g_d_u)
    e_max = jax.ops.segment_max(e_s, sdst, num_segments=N, indices_are_sorted=True)
    g_m_s, g_m_u = _sc_gather([(e_max, sdst_pad), (e_max, dst_pad)])
    e_exp_s = _ew1(_edge_exp_kernel, e_s, g_m_s)
    e_exp_u = _ew1(_edge_exp_kernel, e_u, g_m_u)
    denom = jax.ops.segment_sum(e_exp_u, dst, num_segments=N)
    (g_dn,) = _sc_gather([(denom, sdst_pad)])
    alpha = _ew1(_edge_alpha_kernel, e_exp_s, g_dn)
    upd = _upd(alpha, h[ssrc])
    out = jax.ops.segment_sum(upd, sdst, num_segments=N, indices_are_sorted=True)
    return _act(out, b, act_slope)


def kernel(x, edge_index, W1, a_src1, a_dst1, b1, W2, a_src2, a_dst2, b2):
    loop = jnp.arange(N, dtype=edge_index.dtype)
    src = jnp.concatenate([edge_index[0], loop])
    dst = jnp.concatenate([edge_index[1], loop])
    sdst, ssrc = jax.lax.sort((dst, src), num_keys=1)
    pad = lambda a: jnp.pad(a, (0, EPAD - EP))
    idx = (ssrc, sdst, pad(ssrc), pad(sdst), pad(src), pad(dst), dst)

    x1 = _gat_layer(x, idx, W1, a_src1, a_dst1, b1, 0.01)
    x2 = _gat_layer(x1, idx, W2, a_src2, a_dst2, b2, None)

    value = (x2[edge_index[0]] * x2[edge_index[1]]).sum(axis=1)
    k_homo = int(E * 0.6)
    k_het = E - k_homo
    _, topk_homo = jax.lax.top_k(value, k_homo)
    _, topk_het = jax.lax.top_k(-value, k_het)
    return edge_index[:, topk_homo], edge_index[:, topk_het], x2


# speedup vs baseline: 2.9345x; 1.0579x over previous
import functools

import jax
import jax.numpy as jnp
from jax import lax
from jax.experimental import pallas as pl
from jax.experimental.pallas import tpu as pltpu
from jax.experimental.pallas import tpu_sc as plsc

N = 10000
E = 320000
EP = 330000
NW = 32
EPAD = 330240
BPW = EPAD // NW


def _sc_gather_call(tables, idxs):
    k = len(tables)
    mesh = plsc.VectorSubcoreMesh(core_axis_name="c", subcore_axis_name="s")
    dtypes = [t.dtype for t in tables]

    @functools.partial(
        pl.kernel,
        out_type=tuple(jax.ShapeDtypeStruct((EPAD,), dt) for dt in dtypes),
        mesh=mesh,
        scratch_types=[pltpu.VMEM((BPW,), jnp.int32),
                       pltpu.VMEM((BPW,), jnp.float32),
                       pltpu.SemaphoreType.DMA],
    )
    def _k(*refs):
        tab_hbm = refs[:k]
        idx_hbm = refs[k:2 * k]
        out_hbm = refs[2 * k:3 * k]
        idx_v, out_v, sem = refs[3 * k:]
        wid = lax.axis_index("s") * 2 + lax.axis_index("c")
        base = wid * BPW
        ch = 1032
        for p in range(k):
            pltpu.sync_copy(idx_hbm[p].at[pl.ds(base, BPW)], idx_v)
            cps = [pltpu.async_copy(tab_hbm[p].at[idx_v.at[pl.ds(j * ch, ch)]],
                                    out_v.at[pl.ds(j * ch, ch)], sem)
                   for j in range(BPW // ch)]
            for c in cps:
                c.wait()
            pltpu.sync_copy(out_v, out_hbm[p].at[pl.ds(base, BPW)])

    outs = _k(*tables, *idxs)
    return (outs,) if not isinstance(outs, (tuple, list)) else tuple(outs)


def _sc_gather(pairs):
    tables = [t for t, _ in pairs]
    idxs = [i for _, i in pairs]
    outs = _sc_gather_call(tables, idxs)
    return [o[:EP] for o in outs]


def _mm_kernel(x_ref, w_ref, o_ref):
    o_ref[...] = jnp.dot(x_ref[...], w_ref[...], preferred_element_type=jnp.float32)


def _matmul(x, w):
    return pl.pallas_call(
        _mm_kernel,
        out_shape=jax.ShapeDtypeStruct((x.shape[0], w.shape[1]), jnp.float32),
    )(x, w)


def _edge_e_kernel(s_ref, d_ref, o_ref):
    e = s_ref[...] + d_ref[...]
    o_ref[...] = jnp.where(e > 0, e, 0.2 * e)


def _edge_exp_kernel(e_ref, m_ref, o_ref):
    o_ref[...] = jnp.exp(e_ref[...] - m_ref[...])


def _edge_alpha_kernel(z_ref, dn_ref, o_ref):
    o_ref[...] = z_ref[...] / dn_ref[...]


def _ew1(kernel_fn, a, b):
    n = a.shape[0]
    p = ((n + 1023) // 1024) * 1024
    ap = jnp.pad(a, (0, p - n))
    bp = jnp.pad(b, (0, p - n))
    out = pl.pallas_call(
        kernel_fn, out_shape=jax.ShapeDtypeStruct((p,), jnp.float32)
    )(ap, bp)
    return out[:n]


def _ew2(kernel_fn, a1, b1, a2, b2):
    n = a1.shape[0]
    p = ((n + 1023) // 1024) * 1024
    pads = [jnp.pad(t, (0, p - n)) for t in (a1, b1, a2, b2)]
    o1, o2 = pl.pallas_call(
        kernel_fn,
        out_shape=(jax.ShapeDtypeStruct((p,), jnp.float32),
                   jax.ShapeDtypeStruct((p,), jnp.float32)),
    )(*pads)
    return o1[:n], o2[:n]


def _edge_e2_kernel(s1, d1, s2, d2, o1, o2):
    e1 = s1[...] + d1[...]
    o1[...] = jnp.where(e1 > 0, e1, 0.2 * e1)
    e2 = s2[...] + d2[...]
    o2[...] = jnp.where(e2 > 0, e2, 0.2 * e2)


def _edge_exp2_kernel(e1, m1, e2, m2, o1, o2):
    o1[...] = jnp.exp(e1[...] - m1[...])
    o2[...] = jnp.exp(e2[...] - m2[...])


def _upd_kernel(al_ref, h_ref, o_ref):
    o_ref[...] = al_ref[...][:, None] * h_ref[...]


def _upd(alpha, hsrc):
    n, f = hsrc.shape
    blk = 8192
    grid = (n + blk - 1) // blk
    return pl.pallas_call(
        _upd_kernel,
        grid=(grid,),
        in_specs=[
            pl.BlockSpec((blk,), lambda i: (i,)),
            pl.BlockSpec((blk, f), lambda i: (i, 0)),
        ],
        out_specs=pl.BlockSpec((blk, f), lambda i: (i, 0)),
        out_shape=jax.ShapeDtypeStruct((n, f), jnp.float32),
    )(alpha, hsrc)


def _act_leaky_kernel(slope, o_ref, b_ref, y_ref):
    t = o_ref[...] + b_ref[...]
    y_ref[...] = jnp.where(t > 0, t, slope * t)


def _act_bias_kernel(o_ref, b_ref, y_ref):
    y_ref[...] = o_ref[...] + b_ref[...]


def _act(out, b, slope):
    n, f = out.shape
    if slope is None:
        body = _act_bias_kernel
    else:
        body = functools.partial(_act_leaky_kernel, slope)
    return pl.pallas_call(
        body, out_shape=jax.ShapeDtypeStruct((n, f), jnp.float32)
    )(out, jnp.broadcast_to(b[None, :], (n, f)))


def _gat_layer(x, idx, W, a_src, a_dst, b, act_slope):
    ssrc, sdst, ssrc_pad, sdst_pad, src_pad, dst_pad, dst = idx
    h = _matmul(x, W)
    al_s = (h * a_src).sum(axis=-1)
    al_d = (h * a_dst).sum(axis=-1)
    g_s_s, g_d_s, g_s_u, g_d_u = _sc_gather(
        [(al_s, ssrc_pad), (al_d, sdst_pad), (al_s, src_pad), (al_d, dst_pad)])
    e_s, e_u = _ew2(_edge_e2_kernel, g_s_s, g_d_s, g_s_u, g_d_u)
    e_max = jax.ops.segment_max(e_s, sdst, num_segments=N, indices_are_sorted=True)
    g_m_s, g_m_u = _sc_gather([(e_max, sdst_pad), (e_max, dst_pad)])
    e_exp_s, e_exp_u = _ew2(_edge_exp2_kernel, e_s, g_m_s, e_u, g_m_u)
    denom = jax.ops.segment_sum(e_exp_u, dst, num_segments=N)
    (g_dn,) = _sc_gather([(denom, sdst_pad)])
    alpha = _ew1(_edge_alpha_kernel, e_exp_s, g_dn)
    upd = _upd(alpha, h[ssrc])
    out = jax.ops.segment_sum(upd, sdst, num_segments=N, indices_are_sorted=True)
    return _act(out, b, act_slope)


def kernel(x, edge_index, W1, a_src1, a_dst1, b1, W2, a_src2, a_dst2, b2):
    loop = jnp.arange(N, dtype=edge_index.dtype)
    src = jnp.concatenate([edge_index[0], loop])
    dst = jnp.concatenate([edge_index[1], loop])
    sdst, ssrc = jax.lax.sort((dst, src), num_keys=1)
    pad = lambda a: jnp.pad(a, (0, EPAD - EP))
    idx = (ssrc, sdst, pad(ssrc), pad(sdst), pad(src), pad(dst), dst)

    x1 = _gat_layer(x, idx, W1, a_src1, a_dst1, b1, 0.01)
    x2 = _gat_layer(x1, idx, W2, a_src2, a_dst2, b2, None)

    value = (x2[edge_index[0]] * x2[edge_index[1]]).sum(axis=1)
    k_homo = int(E * 0.6)
    k_het = E - k_homo
    _, topk_homo = jax.lax.top_k(value, k_homo)
    _, topk_het = jax.lax.top_k(-value, k_het)
    return edge_index[:, topk_homo], edge_index[:, topk_het], x2
